# baseline (speedup 1.0000x reference)
import jax
import jax.numpy as jnp
from jax import lax
from jax.experimental import pallas as pl
from jax.experimental.pallas import tpu as pltpu

_F32 = jnp.float32
_BF16 = jnp.bfloat16

_NG = 64
_NN = 2560
_HID = 64
_W = 4 * _HID
_CR = 10
_CRP = 16
_LC = 16
_GH = 128
_NSTEP = 8
_GB = _NG // _NSTEP
_GN = _NN // _NSTEP
_ODIM = 8

_S_GW = (0, 256, 512)
_S_WIE, _S_WHH0, _S_WIH1 = 768, 1024, 1280
_S_WHH1, _S_FW1, _S_FW2 = 1536, 1792, 2048
_S_WGM, _S_MSEL, _S_BIAS = 2304, 2320, 2336


def _body(a_ref, x_ref, p_ref, gm_ref, w_ref, o_ref, emb_s, pre_s, hseq_s):
    i = pl.program_id(0)

    def brow(k, lanes=_W):
        r = _S_BIAS + k
        return w_ref[r:r + 1, :lanes]

    a = a_ref[0]
    h = x_ref[...]
    y = None
    for l in range(3):
        t = jnp.dot(a, h, preferred_element_type=_F32)
        gw = w_ref[_S_GW[l]:_S_GW[l] + _GH, :_GH].astype(_BF16)
        y = jnp.dot(t.astype(_BF16), gw, preferred_element_type=_F32)
        y = y + brow(l, _GH)
        if l < 2:
            y = jnp.maximum(y, 0.0)
        h = y.astype(_BF16)
    emb_s[pl.ds(i * _GB, _GB), :] = jnp.dot(p_ref[0], y,
                                            preferred_element_type=_F32)

    @pl.when(i == _NSTEP - 1)
    def _lstm_and_head():
        gm = gm_ref[...]

        emb_g = (jnp.dot(emb_s[...], w_ref[_S_WIE:_S_WIE + _GH, :],
                         preferred_element_type=_F32) + brow(3))
        pre = jnp.dot(gm, w_ref[_S_WGM:_S_WGM + _CRP, :],
                      preferred_element_type=_F32)
        for t in range(_LC):
            pre_s[t * _NG:(t + 1) * _NG, :] = (
                pre[t * _NG:(t + 1) * _NG, :] + emb_g)

        lane = lax.broadcasted_iota(jnp.int32, (_NG, _W), 1)
        g_sel = (lane >= 2 * _HID) & (lane < 3 * _HID)
        whh0 = w_ref[_S_WHH0:_S_WHH0 + _W, :]
        wih1 = w_ref[_S_WIH1:_S_WIH1 + _W, :]
        whh1 = w_ref[_S_WHH1:_S_WHH1 + _W, :]
        b1 = brow(4)

        def cell(gates, c_old):
            s = jax.nn.sigmoid(jnp.where(g_sel, gates + gates, gates))
            act = jnp.where(g_sel, s + s - 1.0, s)
            f_al = pltpu.roll(act, 3 * _HID, 1)
            g_al = pltpu.roll(act, 2 * _HID, 1)
            o_al = pltpu.roll(act, _HID, 1)
            c_new = f_al * c_old + act * g_al
            h_new = o_al * jnp.tanh(c_new)
            return h_new, c_new

        zeros = jnp.zeros((_NG, _W), _F32)
        h0, c0, h1, c1 = zeros, zeros, zeros, zeros
        for t in range(_LC):
            g0 = (pre_s[t * _NG:(t + 1) * _NG, :]
                  + jnp.dot(h0, whh0, preferred_element_type=_F32))
            h0, c0 = cell(g0, c0)
            g1 = (jnp.dot(h0, wih1, preferred_element_type=_F32)
                  + jnp.dot(h1, whh1, preferred_element_type=_F32) + b1)
            h1, c1 = cell(g1, c1)
            hseq_s[t * _NG:(t + 1) * _NG, :] = h1

        mask = jnp.dot(gm, w_ref[_S_MSEL:_S_MSEL + _CRP, :],
                       preferred_element_type=_F32)
        hm = hseq_s[...] * mask
        yh = jnp.maximum(jnp.dot(hm, w_ref[_S_FW1:_S_FW1 + _W, :],
                                 preferred_element_type=_F32) + brow(5), 0.0)
        o_ref[...] = (jnp.dot(yh, w_ref[_S_FW2:_S_FW2 + _W, :],
                              preferred_element_type=_F32) + brow(6))


def kernel(wslab, x_pad, adj, pool_pad, ground_motion, time_steps):
    f32 = _F32
    gm = ground_motion.reshape(_NG, _LC, _CR).astype(f32)
    comp_len = jnp.floor(time_steps.astype(f32) / _CR)
    mask_bt = (jnp.arange(_LC, dtype=f32)[None, :] < comp_len[:, None]).astype(f32)
    extra = jnp.zeros((_NG, _LC, _CRP - _CR), f32).at[:, :, 0].set(mask_bt)
    gmx = jnp.transpose(jnp.concatenate([gm, extra], axis=2),
                        (1, 0, 2)).reshape(_LC * _NG, _CRP)

    ablk = jnp.stack([adj[k * _GN:(k + 1) * _GN, k * _GN:(k + 1) * _GN]
                      for k in range(_NSTEP)]).astype(_BF16)
    xb = x_pad[:, :_GH].astype(_BF16)
    pblk = jnp.stack([pool_pad[k * _GB:(k + 1) * _GB, k * _GN:(k + 1) * _GN]
                      for k in range(_NSTEP)])

    out = pl.pallas_call(
        _body,
        out_shape=jax.ShapeDtypeStruct((_LC * _NG, _W), f32),
        grid=(_NSTEP,),
        in_specs=[
            pl.BlockSpec((1, _GN, _GN), lambda i: (i, 0, 0)),
            pl.BlockSpec((_GN, _GH), lambda i: (i, 0)),
            pl.BlockSpec((1, _GB, _GN), lambda i: (i, 0, 0)),
            pl.BlockSpec((_LC * _NG, _CRP), lambda i: (0, 0)),
            pl.BlockSpec(wslab.shape, lambda i: (0, 0)),
        ],
        out_specs=pl.BlockSpec((_LC * _NG, _W), lambda i: (0, 0)),
        scratch_shapes=[pltpu.VMEM((_NG, _GH), f32),
                        pltpu.VMEM((_LC * _NG, _W), f32),
                        pltpu.VMEM((_LC * _NG, _W), f32)],
        compiler_params=pltpu.CompilerParams(
            dimension_semantics=("arbitrary",)),
    )(ablk, xb, pblk, gmx, wslab)

    out = out.reshape(_LC, _NG, _W)
    out = jnp.transpose(out, (1, 0, 2))
    return out[:, :, :_ODIM]


# speedup vs baseline: 1.8859x; 1.8859x over previous
import jax
import jax.numpy as jnp
from jax import lax
from jax.experimental import pallas as pl
from jax.experimental.pallas import tpu as pltpu

_F32 = jnp.float32
_BF16 = jnp.bfloat16

_NG = 64
_NN = 2560
_HID = 64
_W = 4 * _HID
_CR = 10
_CRP = 16
_LC = 16
_GH = 128
_NSTEP = 4
_GB = _NG // _NSTEP
_GN = _NN // _NSTEP
_ODIM = 8

_S_GW = (0, 256, 512)
_S_WIE, _S_WHH0, _S_WIH1 = 768, 1024, 1280
_S_WHH1, _S_FW1, _S_FW2 = 1536, 1792, 2048
_S_WGM, _S_MSEL, _S_BIAS = 2304, 2320, 2336


def _body(a_ref, x_ref, p_ref, gm_ref, w_ref, o_ref, emb_s, pre_s, hseq_s):
    i = pl.program_id(0)

    def brow(k, lanes=_W):
        r = _S_BIAS + k
        return w_ref[r:r + 1, :lanes]

    a = a_ref[...].astype(_BF16)
    h = x_ref[...].astype(_BF16)
    y = None
    for l in range(3):
        t = jnp.dot(a, h, preferred_element_type=_F32)
        gw = w_ref[_S_GW[l]:_S_GW[l] + _GH, :_GH].astype(_BF16)
        y = jnp.dot(t.astype(_BF16), gw, preferred_element_type=_F32)
        y = y + brow(l, _GH)
        if l < 2:
            y = jnp.maximum(y, 0.0)
        h = y.astype(_BF16)
    emb_s[pl.ds(i * _GB, _GB), :] = jnp.dot(p_ref[...], y,
                                            preferred_element_type=_F32)

    @pl.when(i == _NSTEP - 1)
    def _lstm_and_head():
        gm = gm_ref[...]

        emb_g = (jnp.dot(emb_s[...], w_ref[_S_WIE:_S_WIE + _GH, :],
                         preferred_element_type=_F32) + brow(3))
        pre = jnp.dot(gm, w_ref[_S_WGM:_S_WGM + _CRP, :],
                      preferred_element_type=_F32)
        for t in range(_LC):
            pre_s[t * _NG:(t + 1) * _NG, :] = (
                pre[t * _NG:(t + 1) * _NG, :] + emb_g)

        lane = lax.broadcasted_iota(jnp.int32, (_NG, _W), 1)
        g_sel = (lane >= 2 * _HID) & (lane < 3 * _HID)
        whh0 = w_ref[_S_WHH0:_S_WHH0 + _W, :]
        wih1 = w_ref[_S_WIH1:_S_WIH1 + _W, :]
        whh1 = w_ref[_S_WHH1:_S_WHH1 + _W, :]
        b1 = brow(4)

        def cell(gates, c_old):
            s = jax.nn.sigmoid(jnp.where(g_sel, gates + gates, gates))
            act = jnp.where(g_sel, s + s - 1.0, s)
            f_al = pltpu.roll(act, 3 * _HID, 1)
            g_al = pltpu.roll(act, 2 * _HID, 1)
            o_al = pltpu.roll(act, _HID, 1)
            c_new = f_al * c_old + act * g_al
            h_new = o_al * jnp.tanh(c_new)
            return h_new, c_new

        zeros = jnp.zeros((_NG, _W), _F32)
        h0, c0, h1, c1 = zeros, zeros, zeros, zeros
        for t in range(_LC):
            g0 = (pre_s[t * _NG:(t + 1) * _NG, :]
                  + jnp.dot(h0, whh0, preferred_element_type=_F32))
            h0, c0 = cell(g0, c0)
            g1 = (jnp.dot(h0, wih1, preferred_element_type=_F32)
                  + jnp.dot(h1, whh1, preferred_element_type=_F32) + b1)
            h1, c1 = cell(g1, c1)
            hseq_s[t * _NG:(t + 1) * _NG, :] = h1

        mask = jnp.dot(gm, w_ref[_S_MSEL:_S_MSEL + _CRP, :],
                       preferred_element_type=_F32)
        hm = hseq_s[...] * mask
        yh = jnp.maximum(jnp.dot(hm, w_ref[_S_FW1:_S_FW1 + _W, :],
                                 preferred_element_type=_F32) + brow(5), 0.0)
        o_ref[...] = (jnp.dot(yh, w_ref[_S_FW2:_S_FW2 + _W, :],
                              preferred_element_type=_F32) + brow(6))


def kernel(wslab, x_pad, adj, pool_pad, ground_motion, time_steps):
    f32 = _F32
    gm = ground_motion.reshape(_NG, _LC, _CR).astype(f32)
    comp_len = jnp.floor(time_steps.astype(f32) / _CR)
    mask_bt = (jnp.arange(_LC, dtype=f32)[None, :] < comp_len[:, None]).astype(f32)
    extra = jnp.zeros((_NG, _LC, _CRP - _CR), f32).at[:, :, 0].set(mask_bt)
    gmx = jnp.transpose(jnp.concatenate([gm, extra], axis=2),
                        (1, 0, 2)).reshape(_LC * _NG, _CRP)

    out = pl.pallas_call(
        _body,
        out_shape=jax.ShapeDtypeStruct((_LC * _NG, _W), f32),
        grid=(_NSTEP,),
        in_specs=[
            pl.BlockSpec((_GN, _GN), lambda i: (i, i)),
            pl.BlockSpec((_GN, _GH), lambda i: (i, 0)),
            pl.BlockSpec((_GB, _GN), lambda i: (i, i)),
            pl.BlockSpec((_LC * _NG, _CRP), lambda i: (0, 0)),
            pl.BlockSpec(wslab.shape, lambda i: (0, 0)),
        ],
        out_specs=pl.BlockSpec((_LC * _NG, _W), lambda i: (0, 0)),
        scratch_shapes=[pltpu.VMEM((_NG, _GH), f32),
                        pltpu.VMEM((_LC * _NG, _W), f32),
                        pltpu.VMEM((_LC * _NG, _W), f32)],
        compiler_params=pltpu.CompilerParams(
            dimension_semantics=("arbitrary",)),
    )(adj, x_pad, pool_pad, gmx, wslab)

    out = out.reshape(_LC, _NG, _W)
    out = jnp.transpose(out, (1, 0, 2))
    return out[:, :, :_ODIM]


# speedup vs baseline: 1.9660x; 1.0425x over previous
import jax
import jax.numpy as jnp
from jax import lax
from jax.experimental import pallas as pl
from jax.experimental.pallas import tpu as pltpu

_F32 = jnp.float32
_BF16 = jnp.bfloat16

_NG = 64
_NN = 2560
_HID = 64
_W = 4 * _HID
_CR = 10
_CRP = 16
_LC = 16
_GH = 128
_NSTEP = 4
_GB = _NG // _NSTEP
_GN = _NN // _NSTEP
_ODIM = 8

_S_GW = (0, 256, 512)
_S_WIE, _S_WHH0, _S_WIH1 = 768, 1024, 1280
_S_WHH1, _S_FW1, _S_FW2 = 1536, 1792, 2048
_S_WGM, _S_MSEL, _S_BIAS = 2304, 2320, 2336


def _body(a_ref, x_ref, p_ref, gm_ref, w_ref, o_ref, emb_s, pre_s, hseq_s):
    i = pl.program_id(0)

    def brow(k, lanes=_W):
        r = _S_BIAS + k
        return w_ref[r:r + 1, :lanes]

    a = a_ref[...].astype(_BF16)
    h = x_ref[...].astype(_BF16)
    y = None
    for l in range(3):
        t = jnp.dot(a, h, preferred_element_type=_F32)
        gw = w_ref[_S_GW[l]:_S_GW[l] + _GH, :_GH].astype(_BF16)
        y = jnp.dot(t.astype(_BF16), gw, preferred_element_type=_F32)
        y = y + brow(l, _GH)
        if l < 2:
            y = jnp.maximum(y, 0.0)
        h = y.astype(_BF16)
    emb_s[pl.ds(i * _GB, _GB), :] = jnp.dot(p_ref[...], y,
                                            preferred_element_type=_F32)

    @pl.when(i == _NSTEP - 1)
    def _lstm_and_head():
        gm = gm_ref[...]

        emb_g = (jnp.dot(emb_s[...], w_ref[_S_WIE:_S_WIE + _GH, :],
                         preferred_element_type=_F32) + brow(3))
        pre = jnp.dot(gm, w_ref[_S_WGM:_S_WGM + _CRP, :],
                      preferred_element_type=_F32)
        for t in range(_LC):
            pre_s[t * _NG:(t + 1) * _NG, :] = (
                pre[t * _NG:(t + 1) * _NG, :] + emb_g)

        lane = lax.broadcasted_iota(jnp.int32, (_NG, _W), 1)
        g_sel = (lane >= 2 * _HID) & (lane < 3 * _HID)
        whh0 = w_ref[_S_WHH0:_S_WHH0 + _W, :]
        wih1 = w_ref[_S_WIH1:_S_WIH1 + _W, :]
        whh1 = w_ref[_S_WHH1:_S_WHH1 + _W, :]
        b1 = brow(4)

        def cell(gates, c_old):
            tt = jnp.tanh(jnp.where(g_sel, gates, 0.5 * gates))
            act = jnp.where(g_sel, tt, 0.5 + 0.5 * tt)
            f_al = pltpu.roll(act, 3 * _HID, 1)
            g_al = pltpu.roll(act, 2 * _HID, 1)
            o_al = pltpu.roll(act, _HID, 1)
            c_new = f_al * c_old + act * g_al
            h_new = o_al * jnp.tanh(c_new)
            return h_new, c_new

        zeros = jnp.zeros((_NG, _W), _F32)
        h0, c0, h1, c1 = zeros, zeros, zeros, zeros
        for t in range(_LC):
            g0 = (pre_s[t * _NG:(t + 1) * _NG, :]
                  + jnp.dot(h0, whh0, preferred_element_type=_F32))
            h0, c0 = cell(g0, c0)
            g1 = (jnp.dot(h0, wih1, preferred_element_type=_F32)
                  + jnp.dot(h1, whh1, preferred_element_type=_F32) + b1)
            h1, c1 = cell(g1, c1)
            hseq_s[t * _NG:(t + 1) * _NG, :] = h1

        mask = jnp.dot(gm, w_ref[_S_MSEL:_S_MSEL + _CRP, :],
                       preferred_element_type=_F32)
        hm = hseq_s[...] * mask
        yh = jnp.maximum(jnp.dot(hm, w_ref[_S_FW1:_S_FW1 + _W, :],
                                 preferred_element_type=_F32) + brow(5), 0.0)
        o_ref[...] = (jnp.dot(yh, w_ref[_S_FW2:_S_FW2 + _W, :_GH],
                              preferred_element_type=_F32) + brow(6, _GH))


def kernel(wslab, x_pad, adj, pool_pad, ground_motion, time_steps):
    f32 = _F32
    gm = ground_motion.reshape(_NG, _LC, _CR).astype(f32)
    comp_len = jnp.floor(time_steps.astype(f32) / _CR)
    mask_bt = (jnp.arange(_LC, dtype=f32)[None, :] < comp_len[:, None]).astype(f32)
    extra = jnp.zeros((_NG, _LC, _CRP - _CR), f32).at[:, :, 0].set(mask_bt)
    gmx = jnp.transpose(jnp.concatenate([gm, extra], axis=2),
                        (1, 0, 2)).reshape(_LC * _NG, _CRP)

    out = pl.pallas_call(
        _body,
        out_shape=jax.ShapeDtypeStruct((_LC * _NG, _GH), f32),
        grid=(_NSTEP,),
        in_specs=[
            pl.BlockSpec((_GN, _GN), lambda i: (i, i)),
            pl.BlockSpec((_GN, _GH), lambda i: (i, 0)),
            pl.BlockSpec((_GB, _GN), lambda i: (i, i)),
            pl.BlockSpec((_LC * _NG, _CRP), lambda i: (0, 0)),
            pl.BlockSpec(wslab.shape, lambda i: (0, 0)),
        ],
        out_specs=pl.BlockSpec((_LC * _NG, _GH), lambda i: (0, 0)),
        scratch_shapes=[pltpu.VMEM((_NG, _GH), f32),
                        pltpu.VMEM((_LC * _NG, _W), f32),
                        pltpu.VMEM((_LC * _NG, _W), f32)],
        compiler_params=pltpu.CompilerParams(
            dimension_semantics=("arbitrary",)),
    )(adj, x_pad, pool_pad, gmx, wslab)

    out = out.reshape(_LC, _NG, _GH)
    out = jnp.transpose(out, (1, 0, 2))
    return out[:, :, :_ODIM]


# speedup vs baseline: 1.9869x; 1.0107x over previous
import jax
import jax.numpy as jnp
from jax import lax
from jax.experimental import pallas as pl
from jax.experimental.pallas import tpu as pltpu

_F32 = jnp.float32
_BF16 = jnp.bfloat16

_NG = 64
_NN = 2560
_HID = 64
_W = 4 * _HID
_CR = 10
_CRP = 16
_LC = 16
_GH = 128
_NSTEP = 4
_GB = _NG // _NSTEP
_GN = _NN // _NSTEP
_ODIM = 8

_S_GW = (0, 256, 512)
_S_WIE, _S_WHH0, _S_WIH1 = 768, 1024, 1280
_S_WHH1, _S_FW1, _S_FW2 = 1536, 1792, 2048
_S_WGM, _S_MSEL, _S_BIAS = 2304, 2320, 2336


def _body(a_ref, x_ref, p_ref, gm_ref, w_ref, o_ref, emb_s, pre_s, hseq_s):
    i = pl.program_id(0)

    def brow(k, lanes=_W):
        r = _S_BIAS + k
        return w_ref[r:r + 1, :lanes]

    a = a_ref[...].astype(_BF16)
    h = x_ref[...].astype(_BF16)
    y = None
    for l in range(3):
        t = jnp.dot(a, h, preferred_element_type=_F32)
        gw = w_ref[_S_GW[l]:_S_GW[l] + _GH, :_GH].astype(_BF16)
        y = jnp.dot(t.astype(_BF16), gw, preferred_element_type=_F32)
        y = y + brow(l, _GH)
        if l < 2:
            y = jnp.maximum(y, 0.0)
        h = y.astype(_BF16)
    emb_s[pl.ds(i * _GB, _GB), :] = jnp.dot(p_ref[...], y,
                                            preferred_element_type=_F32)

    @pl.when(i == _NSTEP - 1)
    def _lstm_and_head():
        gm = gm_ref[...]

        emb_g = (jnp.dot(emb_s[...], w_ref[_S_WIE:_S_WIE + _GH, :],
                         preferred_element_type=_F32) + brow(3))
        pre = jnp.dot(gm, w_ref[_S_WGM:_S_WGM + _CRP, :],
                      preferred_element_type=_F32)
        for t in range(_LC):
            pre_s[t * _NG:(t + 1) * _NG, :] = (
                pre[t * _NG:(t + 1) * _NG, :] + emb_g)

        lane = lax.broadcasted_iota(jnp.int32, (_NG, _W), 1)
        g_sel = (lane >= 2 * _HID) & (lane < 3 * _HID)
        whh0 = w_ref[_S_WHH0:_S_WHH0 + _HID, :]
        wih1 = w_ref[_S_WIH1:_S_WIH1 + _HID, :]
        whh1 = w_ref[_S_WHH1:_S_WHH1 + _HID, :]
        b1 = brow(4)

        def cell(gates, c_old):
            tt = jnp.tanh(jnp.where(g_sel, gates, 0.5 * gates))
            act = jnp.where(g_sel, tt, 0.5 + 0.5 * tt)
            f_al = pltpu.roll(act, 3 * _HID, 1)
            g_al = pltpu.roll(act, 2 * _HID, 1)
            o_al = pltpu.roll(act, _HID, 1)
            c_new = f_al * c_old + act * g_al
            h_new = o_al * jnp.tanh(c_new)
            return h_new, c_new

        zeros = jnp.zeros((_NG, _W), _F32)
        h0, c0, h1, c1 = zeros, zeros, zeros, zeros
        for t in range(_LC):
            g0 = (pre_s[t * _NG:(t + 1) * _NG, :]
                  + jnp.dot(h0[:, :_HID], whh0, preferred_element_type=_F32))
            h0, c0 = cell(g0, c0)
            g1 = (jnp.dot(h0[:, :_HID], wih1, preferred_element_type=_F32)
                  + jnp.dot(h1[:, :_HID], whh1, preferred_element_type=_F32)
                  + b1)
            h1, c1 = cell(g1, c1)
            hseq_s[t * _NG:(t + 1) * _NG, :] = h1[:, :_GH]

        mask = jnp.dot(gm, w_ref[_S_MSEL:_S_MSEL + _CRP, :_GH],
                       preferred_element_type=_F32)
        hm = hseq_s[...] * mask
        yh = jnp.maximum(
            jnp.dot(hm, w_ref[_S_FW1:_S_FW1 + _GH, :_GH],
                    preferred_element_type=_F32) + brow(5, _GH), 0.0)
        o_ref[...] = (jnp.dot(yh, w_ref[_S_FW2:_S_FW2 + _GH, :_GH],
                              preferred_element_type=_F32) + brow(6, _GH))


def kernel(wslab, x_pad, adj, pool_pad, ground_motion, time_steps):
    f32 = _F32
    gm = ground_motion.reshape(_NG, _LC, _CR).astype(f32)
    comp_len = jnp.floor(time_steps.astype(f32) / _CR)
    mask_bt = (jnp.arange(_LC, dtype=f32)[None, :] < comp_len[:, None]).astype(f32)
    extra = jnp.zeros((_NG, _LC, _CRP - _CR), f32).at[:, :, 0].set(mask_bt)
    gmx = jnp.transpose(jnp.concatenate([gm, extra], axis=2),
                        (1, 0, 2)).reshape(_LC * _NG, _CRP)

    out = pl.pallas_call(
        _body,
        out_shape=jax.ShapeDtypeStruct((_LC * _NG, _GH), f32),
        grid=(_NSTEP,),
        in_specs=[
            pl.BlockSpec((_GN, _GN), lambda i: (i, i)),
            pl.BlockSpec((_GN, _GH), lambda i: (i, 0)),
            pl.BlockSpec((_GB, _GN), lambda i: (i, i)),
            pl.BlockSpec((_LC * _NG, _CRP), lambda i: (0, 0)),
            pl.BlockSpec(wslab.shape, lambda i: (0, 0)),
        ],
        out_specs=pl.BlockSpec((_LC * _NG, _GH), lambda i: (0, 0)),
        scratch_shapes=[pltpu.VMEM((_NG, _GH), f32),
                        pltpu.VMEM((_LC * _NG, _W), f32),
                        pltpu.VMEM((_LC * _NG, _GH), f32)],
        compiler_params=pltpu.CompilerParams(
            dimension_semantics=("arbitrary",)),
    )(adj, x_pad, pool_pad, gmx, wslab)

    out = out.reshape(_LC, _NG, _GH)
    out = jnp.transpose(out, (1, 0, 2))
    return out[:, :, :_ODIM]


# speedup vs baseline: 1.9937x; 1.0034x over previous
import jax
import jax.numpy as jnp
from jax import lax
from jax.experimental import pallas as pl
from jax.experimental.pallas import tpu as pltpu

_F32 = jnp.float32
_BF16 = jnp.bfloat16

_NG = 64
_NN = 2560
_HID = 64
_W = 4 * _HID
_CR = 10
_CRP = 16
_LC = 16
_GH = 128
_NSTEP = 4
_GB = _NG // _NSTEP
_GN = _NN // _NSTEP
_ODIM = 8

_S_GW = (0, 256, 512)
_S_WIE, _S_WHH0, _S_WIH1 = 768, 1024, 1280
_S_WHH1, _S_FW1, _S_FW2 = 1536, 1792, 2048
_S_WGM, _S_MSEL, _S_BIAS = 2304, 2320, 2336


def _body(a_ref, x_ref, p_ref, gm_ref, w_ref, o_ref, emb_s, pre_s, hseq_s):
    i = pl.program_id(0)

    def brow(k, lanes=_W):
        r = _S_BIAS + k
        return w_ref[r:r + 1, :lanes]

    a = a_ref[...]
    h = x_ref[...]
    y = None
    for l in range(3):
        t = jnp.dot(a, h, preferred_element_type=_F32)
        gw = w_ref[_S_GW[l]:_S_GW[l] + _GH, :_GH]
        y = jnp.dot(t, gw, preferred_element_type=_F32)
        y = y + brow(l, _GH)
        if l < 2:
            y = jnp.maximum(y, 0.0)
        h = y
    emb_s[pl.ds(i * _GB, _GB), :] = jnp.dot(p_ref[...], y,
                                            preferred_element_type=_F32)

    @pl.when(i == _NSTEP - 1)
    def _lstm_and_head():
        gm = gm_ref[...]

        emb_g = (jnp.dot(emb_s[...], w_ref[_S_WIE:_S_WIE + _GH, :],
                         preferred_element_type=_F32) + brow(3))
        pre = jnp.dot(gm, w_ref[_S_WGM:_S_WGM + _CRP, :],
                      preferred_element_type=_F32)
        for t in range(_LC):
            pre_s[t * _NG:(t + 1) * _NG, :] = (
                pre[t * _NG:(t + 1) * _NG, :] + emb_g)

        lane = lax.broadcasted_iota(jnp.int32, (_NG, _W), 1)
        g_sel = (lane >= 2 * _HID) & (lane < 3 * _HID)
        whh0 = w_ref[_S_WHH0:_S_WHH0 + _HID, :]
        wih1 = w_ref[_S_WIH1:_S_WIH1 + _HID, :]
        whh1 = w_ref[_S_WHH1:_S_WHH1 + _HID, :]
        b1 = brow(4)

        def cell(gates, c_old):
            tt = jnp.tanh(jnp.where(g_sel, gates, 0.5 * gates))
            act = jnp.where(g_sel, tt, 0.5 + 0.5 * tt)
            f_al = pltpu.roll(act, 3 * _HID, 1)
            g_al = pltpu.roll(act, 2 * _HID, 1)
            o_al = pltpu.roll(act, _HID, 1)
            c_new = f_al * c_old + act * g_al
            h_new = o_al * jnp.tanh(c_new)
            return h_new, c_new

        zeros = jnp.zeros((_NG, _W), _F32)
        h0, c0, h1, c1 = zeros, zeros, zeros, zeros
        for t in range(_LC):
            g0 = (pre_s[t * _NG:(t + 1) * _NG, :]
                  + jnp.dot(h0[:, :_HID], whh0, preferred_element_type=_F32))
            h0, c0 = cell(g0, c0)
            g1 = (jnp.dot(h0[:, :_HID], wih1, preferred_element_type=_F32)
                  + jnp.dot(h1[:, :_HID], whh1, preferred_element_type=_F32)
                  + b1)
            h1, c1 = cell(g1, c1)
            hseq_s[t * _NG:(t + 1) * _NG, :] = h1[:, :_GH]

        mask = jnp.dot(gm, w_ref[_S_MSEL:_S_MSEL + _CRP, :_GH],
                       preferred_element_type=_F32)
        hm = hseq_s[...] * mask
        yh = jnp.maximum(
            jnp.dot(hm, w_ref[_S_FW1:_S_FW1 + _GH, :_GH],
                    preferred_element_type=_F32) + brow(5, _GH), 0.0)
        o_ref[...] = (jnp.dot(yh, w_ref[_S_FW2:_S_FW2 + _GH, :_GH],
                              preferred_element_type=_F32) + brow(6, _GH))


def kernel(wslab, x_pad, adj, pool_pad, ground_motion, time_steps):
    f32 = _F32
    gm = ground_motion.reshape(_NG, _LC, _CR).astype(f32)
    comp_len = jnp.floor(time_steps.astype(f32) / _CR)
    mask_bt = (jnp.arange(_LC, dtype=f32)[None, :] < comp_len[:, None]).astype(f32)
    extra = jnp.zeros((_NG, _LC, _CRP - _CR), f32).at[:, :, 0].set(mask_bt)
    gmx = jnp.transpose(jnp.concatenate([gm, extra], axis=2),
                        (1, 0, 2)).reshape(_LC * _NG, _CRP)

    out = pl.pallas_call(
        _body,
        out_shape=jax.ShapeDtypeStruct((_LC * _NG, _GH), f32),
        grid=(_NSTEP,),
        in_specs=[
            pl.BlockSpec((_GN, _GN), lambda i: (i, i)),
            pl.BlockSpec((_GN, _GH), lambda i: (i, 0)),
            pl.BlockSpec((_GB, _GN), lambda i: (i, i)),
            pl.BlockSpec((_LC * _NG, _CRP), lambda i: (0, 0)),
            pl.BlockSpec(wslab.shape, lambda i: (0, 0)),
        ],
        out_specs=pl.BlockSpec((_LC * _NG, _GH), lambda i: (0, 0)),
        scratch_shapes=[pltpu.VMEM((_NG, _GH), f32),
                        pltpu.VMEM((_LC * _NG, _W), f32),
                        pltpu.VMEM((_LC * _NG, _GH), f32)],
        compiler_params=pltpu.CompilerParams(
            dimension_semantics=("arbitrary",)),
    )(adj, x_pad, pool_pad, gmx, wslab)

    out = out.reshape(_LC, _NG, _GH)
    out = jnp.transpose(out, (1, 0, 2))
    return out[:, :, :_ODIM]
